# Initial kernel scaffold; baseline (speedup 1.0000x reference)
import jax
import jax.numpy as jnp
from jax.experimental import pallas as pl


def kernel(data, lengths, embed_init):
    raise NotImplementedError("write your pallas kernel here")


# speedup vs baseline: 5.8335x; 5.8335x over previous
import numpy as np
import jax
import jax.numpy as jnp
from jax.experimental import pallas as pl

_MAX_LEN = 2048
_BATCH = 16
_NUM_AA = 21
_FLAT = _BATCH * _NUM_AA

_A_OF_J = np.tile(np.arange(_NUM_AA, dtype=np.int32), _BATCH).reshape(1, _FLAT)
_B_OF_J = np.repeat(np.arange(_BATCH, dtype=np.int32), _NUM_AA).reshape(1, _FLAT)


def _onehot_mask_kernel(data_ref, aoj_ref, boj_ref, lenf_ref, out_ref):
    data = data_ref[...]
    aoj = aoj_ref[...]
    boj = boj_ref[...]
    lenf = lenf_ref[...]
    d = jnp.broadcast_to(data[:, 0:1], (_MAX_LEN, _FLAT))
    for b in range(1, _BATCH):
        d = jnp.where(boj == b, data[:, b : b + 1], d)
    onehot = d == aoj
    t = jax.lax.broadcasted_iota(jnp.int32, (_MAX_LEN, _FLAT), 0)
    out_ref[...] = jnp.where(onehot & (t < lenf), 1.0, 0.0).astype(jnp.float32)


def kernel(data, lengths, embed_init):
    del embed_init
    lenf = jnp.repeat(lengths.astype(jnp.int32), _NUM_AA).reshape(1, _FLAT)
    out_flat = pl.pallas_call(
        _onehot_mask_kernel,
        out_shape=jax.ShapeDtypeStruct((_MAX_LEN, _FLAT), jnp.float32),
    )(data, jnp.asarray(_A_OF_J), jnp.asarray(_B_OF_J), lenf)
    return out_flat.reshape(_MAX_LEN, _BATCH, _NUM_AA)


# speedup vs baseline: 7.4561x; 1.2781x over previous
import numpy as np
import jax
import jax.numpy as jnp
from jax.experimental import pallas as pl

_MAX_LEN = 2048
_BATCH = 16
_NUM_AA = 21
_FLAT = _BATCH * _NUM_AA
_BLOCK_T = 256

_A_OF_J = np.tile(np.arange(_NUM_AA, dtype=np.float32), _BATCH).reshape(1, _FLAT)
_S = (np.arange(_BATCH)[:, None] == (np.arange(_FLAT)[None, :] // _NUM_AA)).astype(
    np.float32
)


def _onehot_mask_kernel(data_ref, len_ref, aoj_ref, s_ref, out_ref):
    data = data_ref[...]
    lens = len_ref[...]
    base = pl.program_id(0) * _BLOCK_T
    t = base + jax.lax.broadcasted_iota(jnp.int32, (_BLOCK_T, _BATCH), 0)
    q = jnp.where(t < lens, data, _NUM_AA).astype(jnp.float32)
    q336 = jnp.dot(q, s_ref[...], preferred_element_type=jnp.float32)
    out_ref[...] = (q336 == aoj_ref[...]).astype(jnp.float32)


def kernel(data, lengths, embed_init):
    del embed_init
    grid = _MAX_LEN // _BLOCK_T
    out_flat = pl.pallas_call(
        _onehot_mask_kernel,
        grid=(grid,),
        in_specs=[
            pl.BlockSpec((_BLOCK_T, _BATCH), lambda i: (i, 0)),
            pl.BlockSpec((1, _BATCH), lambda i: (0, 0)),
            pl.BlockSpec((1, _FLAT), lambda i: (0, 0)),
            pl.BlockSpec((_BATCH, _FLAT), lambda i: (0, 0)),
        ],
        out_specs=pl.BlockSpec((_BLOCK_T, _FLAT), lambda i: (i, 0)),
        out_shape=jax.ShapeDtypeStruct((_MAX_LEN, _FLAT), jnp.float32),
    )(
        data,
        lengths.astype(jnp.int32).reshape(1, _BATCH),
        jnp.asarray(_A_OF_J),
        jnp.asarray(_S),
    )
    return out_flat.reshape(_MAX_LEN, _BATCH, _NUM_AA)


# speedup vs baseline: 22.6689x; 3.0403x over previous
import jax
import jax.numpy as jnp
from jax.experimental import pallas as pl

_MAX_LEN = 2048
_BATCH = 16
_NUM_AA = 21
_ROWS = _NUM_AA * _BATCH
_AA_PER_BLOCK = 3
_BLOCK_R = _AA_PER_BLOCK * _BATCH


def _onehot_kernel(dataT_ref, len_ref, out_ref):
    dataT = dataT_ref[...]
    lens = len_ref[...]
    t = jax.lax.broadcasted_iota(jnp.int32, (_BATCH, _MAX_LEN), 1)
    q = jnp.where(t < lens, dataT, _NUM_AA)
    a0 = pl.program_id(0) * _AA_PER_BLOCK
    for k in range(_AA_PER_BLOCK):
        out_ref[k * _BATCH : (k + 1) * _BATCH, :] = (q == a0 + k).astype(
            jnp.float32
        )


def kernel(data, lengths, embed_init):
    del embed_init
    dataT = jnp.swapaxes(data, 0, 1)
    out_phys = pl.pallas_call(
        _onehot_kernel,
        grid=(_NUM_AA // _AA_PER_BLOCK,),
        in_specs=[
            pl.BlockSpec((_BATCH, _MAX_LEN), lambda i: (0, 0)),
            pl.BlockSpec((_BATCH, 1), lambda i: (0, 0)),
        ],
        out_specs=pl.BlockSpec((_BLOCK_R, _MAX_LEN), lambda i: (i, 0)),
        out_shape=jax.ShapeDtypeStruct((_ROWS, _MAX_LEN), jnp.float32),
    )(dataT, lengths.astype(jnp.int32).reshape(_BATCH, 1))
    return jnp.transpose(out_phys.reshape(_NUM_AA, _BATCH, _MAX_LEN), (2, 1, 0))
